# baseline (speedup 1.0000x reference)
import math

import jax
import jax.numpy as jnp
from jax.experimental import pallas as pl
from jax.experimental.pallas import tpu as pltpu

T = 2048
D = 768
F = 2048
E = 64
CAP = 64
CHUNK = 128


def _route_kernel(xf_ref, rw_ref, rb_ref, tok_ref, nval_ref):
    xf = xf_ref[...]
    logits = jax.lax.dot_general(
        xf, rw_ref[...], (((1,), (1,)), ((), ())),
        preferred_element_type=jnp.float32)
    logits = logits + rb_ref[...]
    m = jnp.max(logits, axis=1, keepdims=True)
    p = jnp.exp(logits - m)
    probs = p / jnp.sum(p, axis=1, keepdims=True)
    e_col = jnp.argmax(probs, axis=1).astype(jnp.int32)[:, None]

    lane_e = jax.lax.broadcasted_iota(jnp.int32, (T, E), 1)
    oh = (lane_e == e_col).astype(jnp.float32)

    tril = (jax.lax.broadcasted_iota(jnp.int32, (CHUNK, CHUNK), 0)
            > jax.lax.broadcasted_iota(jnp.int32, (CHUNK, CHUNK), 1)
            ).astype(jnp.float32)
    base = jnp.zeros((1, E), jnp.float32)
    rank_rows = []
    for k in range(T // CHUNK):
        chunk = jax.lax.slice(oh, (k * CHUNK, 0), ((k + 1) * CHUNK, E))
        within = jax.lax.dot_general(
            tril, chunk, (((1,), (0,)), ((), ())),
            preferred_element_type=jnp.float32)
        rank_rows.append(within + base)
        base = base + jnp.sum(chunk, axis=0, keepdims=True)
    rank_te = jnp.concatenate(rank_rows, axis=0)
    counts = base

    r = jnp.sum(rank_te * oh, axis=1).astype(jnp.int32)[:, None]
    r = jnp.minimum(r, CAP - 1)

    lane_r = jax.lax.broadcasted_iota(jnp.int32, (T, CAP), 1)
    rankhot = (lane_r == r).astype(jnp.float32)
    ids = jax.lax.broadcasted_iota(jnp.int32, (T, 1), 0).astype(jnp.float32)
    bmat = rankhot * ids
    tok_f = jax.lax.dot_general(
        oh, bmat, (((0,), (0,)), ((), ())),
        preferred_element_type=jnp.float32,
        precision=jax.lax.Precision.HIGHEST)
    tok_ref[...] = jnp.clip(tok_f, 0.0, float(T - 1)).astype(jnp.int32)

    keep = (counts <= float(CAP)).astype(jnp.float32)
    nval_ref[...] = (counts * keep).astype(jnp.int32)


def _ffn_kernel(tok_ref, nval_ref, xf_ref, w1_ref, b1_ref, w2_ref, b2_ref,
                out_ref, xg_ref):
    e = pl.program_id(0)

    @pl.when(e == 0)
    def _init():
        out_ref[...] = jnp.zeros_like(out_ref)

    for i in range(CAP):
        t = jnp.clip(tok_ref[e, i], 0, T - 1)
        xg_ref[pl.ds(i, 1), :] = xf_ref[pl.ds(t, 1), :]

    h = jax.lax.dot_general(
        xg_ref[...], w1_ref[0], (((1,), (0,)), ((), ())),
        preferred_element_type=jnp.float32,
        precision=jax.lax.Precision.HIGHEST)
    h = h + b1_ref[0]
    h = 0.5 * h * (1.0 + jax.lax.erf(h * (1.0 / math.sqrt(2.0))))
    y = jax.lax.dot_general(
        h, w2_ref[0], (((1,), (0,)), ((), ())),
        preferred_element_type=jnp.float32,
        precision=jax.lax.Precision.HIGHEST)
    y = y + b2_ref[0]

    n = nval_ref[0, e]
    for i in range(CAP):
        @pl.when(i < n)
        def _store(i=i):
            t = jnp.clip(tok_ref[e, i], 0, T - 1)
            out_ref[pl.ds(t, 1), :] = y[i:i + 1, :]


def kernel(x, router_W, router_b, W1, b1, W2, b2):
    xf = x.reshape(T, D)
    rb = router_b.reshape(1, E)

    tok, nval = pl.pallas_call(
        _route_kernel,
        out_shape=(
            jax.ShapeDtypeStruct((E, CAP), jnp.int32),
            jax.ShapeDtypeStruct((1, E), jnp.int32),
        ),
    )(xf, router_W, rb)

    out = pl.pallas_call(
        _ffn_kernel,
        grid=(E,),
        in_specs=[
            pl.BlockSpec(memory_space=pltpu.SMEM),
            pl.BlockSpec(memory_space=pltpu.SMEM),
            pl.BlockSpec((T, D), lambda e: (0, 0)),
            pl.BlockSpec((1, D, F), lambda e: (e, 0, 0)),
            pl.BlockSpec((1, 1, F), lambda e: (e, 0, 0)),
            pl.BlockSpec((1, F, D), lambda e: (e, 0, 0)),
            pl.BlockSpec((1, 1, D), lambda e: (e, 0, 0)),
        ],
        out_specs=pl.BlockSpec((T, D), lambda e: (0, 0)),
        out_shape=jax.ShapeDtypeStruct((T, D), jnp.float32),
        scratch_shapes=[pltpu.VMEM((CAP, D), jnp.float32)],
        compiler_params=pltpu.CompilerParams(
            dimension_semantics=("arbitrary",),
            vmem_limit_bytes=100 * 1024 * 1024,
        ),
    )(tok, nval, xf, W1, b1.reshape(E, 1, F), W2, b2.reshape(E, 1, D))

    return out.reshape(x.shape)


# speedup vs baseline: 2.0246x; 2.0246x over previous
import math

import jax
import jax.numpy as jnp
from jax.experimental import pallas as pl
from jax.experimental.pallas import tpu as pltpu

T = 2048
D = 768
F = 2048
E = 64
CAP = 64
CHUNK = 128


def _route_kernel(xf_ref, rw_ref, rb_ref, tok_ref, nval_ref):
    xf = xf_ref[...]
    logits = jax.lax.dot_general(
        xf, rw_ref[...], (((1,), (1,)), ((), ())),
        preferred_element_type=jnp.float32)
    logits = logits + rb_ref[...]
    m = jnp.max(logits, axis=1, keepdims=True)
    p = jnp.exp(logits - m)
    probs = p / jnp.sum(p, axis=1, keepdims=True)
    e_col = jnp.argmax(probs, axis=1).astype(jnp.int32)[:, None]

    lane_e = jax.lax.broadcasted_iota(jnp.int32, (T, E), 1)
    oh = (lane_e == e_col).astype(jnp.float32)

    tril = (jax.lax.broadcasted_iota(jnp.int32, (CHUNK, CHUNK), 0)
            > jax.lax.broadcasted_iota(jnp.int32, (CHUNK, CHUNK), 1)
            ).astype(jnp.float32)
    base = jnp.zeros((1, E), jnp.float32)
    rank_rows = []
    for k in range(T // CHUNK):
        chunk = jax.lax.slice(oh, (k * CHUNK, 0), ((k + 1) * CHUNK, E))
        within = jax.lax.dot_general(
            tril, chunk, (((1,), (0,)), ((), ())),
            preferred_element_type=jnp.float32)
        rank_rows.append(within + base)
        base = base + jnp.sum(chunk, axis=0, keepdims=True)
    rank_te = jnp.concatenate(rank_rows, axis=0)
    counts = base

    r = jnp.sum(rank_te * oh, axis=1).astype(jnp.int32)[:, None]
    r = jnp.minimum(r, CAP - 1)

    lane_r = jax.lax.broadcasted_iota(jnp.int32, (T, CAP), 1)
    rankhot = (lane_r == r).astype(jnp.float32)
    ids = jax.lax.broadcasted_iota(jnp.int32, (T, 1), 0).astype(jnp.float32)
    bmat = rankhot * ids
    tok_f = jax.lax.dot_general(
        oh, bmat, (((0,), (0,)), ((), ())),
        preferred_element_type=jnp.float32,
        precision=jax.lax.Precision.HIGHEST)
    tok_ref[...] = jnp.clip(tok_f, 0.0, float(T - 1)).astype(jnp.int32)

    keep = (counts <= float(CAP)).astype(jnp.float32)
    nval_ref[...] = (counts * keep).astype(jnp.int32)


def _ffn_kernel(tok_ref, nval_ref, xf_ref, w1_ref, b1_ref, w2_ref, b2_ref,
                out_ref, xg_ref):
    e = pl.program_id(0)

    @pl.when(e == 0)
    def _init():
        out_ref[...] = jnp.zeros_like(out_ref)

    for i in range(CAP):
        t = jnp.clip(tok_ref[e, i], 0, T - 1)
        xg_ref[pl.ds(i, 1), :] = xf_ref[pl.ds(t, 1), :]

    h = jax.lax.dot_general(
        xg_ref[...], w1_ref[0], (((1,), (0,)), ((), ())),
        preferred_element_type=jnp.float32)
    h = h + b1_ref[0]
    h = 0.5 * h * (1.0 + jax.lax.erf(h * (1.0 / math.sqrt(2.0))))
    y = jax.lax.dot_general(
        h, w2_ref[0], (((1,), (0,)), ((), ())),
        preferred_element_type=jnp.float32)
    y = y + b2_ref[0]

    n = nval_ref[0, e]
    for i in range(CAP):
        @pl.when(i < n)
        def _store(i=i):
            t = jnp.clip(tok_ref[e, i], 0, T - 1)
            out_ref[pl.ds(t, 1), :] = y[i:i + 1, :]


def kernel(x, router_W, router_b, W1, b1, W2, b2):
    xf = x.reshape(T, D)
    rb = router_b.reshape(1, E)

    tok, nval = pl.pallas_call(
        _route_kernel,
        out_shape=(
            jax.ShapeDtypeStruct((E, CAP), jnp.int32),
            jax.ShapeDtypeStruct((1, E), jnp.int32),
        ),
    )(xf, router_W, rb)

    out = pl.pallas_call(
        _ffn_kernel,
        grid=(E,),
        in_specs=[
            pl.BlockSpec(memory_space=pltpu.SMEM),
            pl.BlockSpec(memory_space=pltpu.SMEM),
            pl.BlockSpec((T, D), lambda e: (0, 0)),
            pl.BlockSpec((1, D, F), lambda e: (e, 0, 0)),
            pl.BlockSpec((1, 1, F), lambda e: (e, 0, 0)),
            pl.BlockSpec((1, F, D), lambda e: (e, 0, 0)),
            pl.BlockSpec((1, 1, D), lambda e: (e, 0, 0)),
        ],
        out_specs=pl.BlockSpec((T, D), lambda e: (0, 0)),
        out_shape=jax.ShapeDtypeStruct((T, D), jnp.float32),
        scratch_shapes=[pltpu.VMEM((CAP, D), jnp.float32)],
        compiler_params=pltpu.CompilerParams(
            dimension_semantics=("arbitrary",),
            vmem_limit_bytes=100 * 1024 * 1024,
        ),
    )(tok, nval, xf, W1, b1.reshape(E, 1, F), W2, b2.reshape(E, 1, D))

    return out.reshape(x.shape)
